# Initial kernel scaffold; baseline (speedup 1.0000x reference)
import jax
import jax.numpy as jnp
from jax.experimental import pallas as pl


def kernel(cand_feat, set_feat, cand_to_set_idx, Wvc, bvc, Wvs, bvs, g_set, b_set, g_cand, b_cand, W1, b1, W2, b2, g_ff, b_ff):
    raise NotImplementedError("write your pallas kernel here")


# speedup vs baseline: 1.1863x; 1.1863x over previous
import functools

import jax
import jax.numpy as jnp
from jax import lax
from jax.experimental import pallas as pl
from jax.experimental.pallas import tpu as pltpu
from jax.experimental.pallas import tpu_sc as plsc

B, C, S, D = 256, 729, 243, 128
NC, NS = 2, 16
NW = NC * NS
BPW = B // NW
SCH = 81
NSCH = C // SCH
GCH = 27
NGCH = C // GCH
L = 16

_mesh = plsc.VectorSubcoreMesh(core_axis_name="c", subcore_axis_name="s",
                               num_cores=NC, num_subcores=NS)
_sc_params = pltpu.CompilerParams(use_tc_tiling_on_sc=False)


def _sc_scatter_body(cand_hbm, sidx_hbm, zeros_hbm, out_hbm, idx_v, src_v, acc_sh):
    cid = lax.axis_index("c")
    sid = lax.axis_index("s")
    wid = sid * NC + cid
    pltpu.sync_copy(sidx_hbm.at[sid], idx_v)

    def batch_body(bi, carry):
        b = wid * BPW + bi
        pltpu.sync_copy(zeros_hbm, acc_sh.at[pl.ds(sid * S, S)])
        for ch in range(NSCH):
            pltpu.sync_copy(cand_hbm.at[b, pl.ds(ch * SCH, SCH)], src_v)
            for j in range(3):
                pltpu.sync_copy(src_v, acc_sh.at[idx_v.at[j, ch]], add=True)
        pltpu.sync_copy(acc_sh.at[pl.ds(sid * S, S)], out_hbm.at[b])
        return carry

    lax.fori_loop(0, BPW, batch_body, 0)


_sc_scatter = pl.kernel(
    _sc_scatter_body,
    out_type=jax.ShapeDtypeStruct((B, S, D), jnp.float32),
    mesh=_mesh,
    compiler_params=_sc_params,
    scratch_types=[
        pltpu.VMEM((3, NSCH, SCH), jnp.int32),
        pltpu.VMEM((SCH, D), jnp.float32),
        pltpu.VMEM_SHARED((NS * S, D), jnp.float32),
    ],
)


def _sc_gather_body(proj_hbm, gidx_hbm, out_hbm, idx_v, gbuf_v, obuf_v):
    cid = lax.axis_index("c")
    sid = lax.axis_index("s")
    wid = sid * NC + cid

    def batch_body(bi, carry):
        b = wid * BPW + bi
        pltpu.sync_copy(gidx_hbm.at[b], idx_v)

        def chunk_body(ch, c2):
            pltpu.sync_copy(proj_hbm.at[idx_v.at[ch]], gbuf_v)
            for i in range(GCH):
                for k in range(D // L):
                    sl = pl.ds(k * L, L)
                    obuf_v[i, sl] = (gbuf_v[3 * i, sl] + gbuf_v[3 * i + 1, sl]
                                     + gbuf_v[3 * i + 2, sl])
            pltpu.sync_copy(obuf_v, out_hbm.at[b, pl.ds(ch * GCH, GCH)])
            return c2

        lax.fori_loop(0, NGCH, chunk_body, 0)
        return carry

    lax.fori_loop(0, BPW, batch_body, 0)


_sc_gather = pl.kernel(
    _sc_gather_body,
    out_type=jax.ShapeDtypeStruct((B, C, D), jnp.float32),
    mesh=_mesh,
    compiler_params=_sc_params,
    scratch_types=[
        pltpu.VMEM((NGCH, 3 * GCH), jnp.int32),
        pltpu.VMEM((3 * GCH, D), jnp.float32),
        pltpu.VMEM((GCH, D), jnp.float32),
    ],
)


def _ln(x, g, b, eps=1e-5):
    m = x.mean(-1, keepdims=True)
    v = ((x - m) ** 2).mean(-1, keepdims=True)
    return (x - m) / jnp.sqrt(v + eps) * g + b


_MID_BB = 4


def _tc_mid_body(acc_ref, sf_ref, wvc_ref, dbvc_ref, gs_ref, bs_ref, wvs_ref,
                 bvs_ref, so_ref, pr_ref):
    wvc = wvc_ref[...]
    wvs = wvs_ref[...]
    dbvc = dbvc_ref[...]
    for i in range(_MID_BB):
        su = jnp.dot(acc_ref[i], wvc, preferred_element_type=jnp.float32) * (1.0 / 9.0)
        x = sf_ref[i] + su + dbvc
        so = _ln(x, gs_ref[...], bs_ref[...])
        so_ref[i] = so
        pr_ref[i] = (jnp.dot(so, wvs, preferred_element_type=jnp.float32) * (1.0 / 3.0)
                     + bvs_ref[...] * (1.0 / 3.0))


def _tc_mid(acc, set_feat, Wvc, dbvc, g_set, b_set, Wvs, bvs):
    grid = (B // _MID_BB,)
    bs3 = pl.BlockSpec((_MID_BB, S, D), lambda i: (i, 0, 0))
    full2 = pl.BlockSpec((S, D), lambda i: (0, 0))
    sq = pl.BlockSpec((D, D), lambda i: (0, 0))
    vec = pl.BlockSpec((D,), lambda i: (0,))
    return pl.pallas_call(
        _tc_mid_body,
        grid=grid,
        in_specs=[bs3, bs3, sq, full2, vec, vec, sq, vec],
        out_specs=[bs3, bs3],
        out_shape=[jax.ShapeDtypeStruct((B, S, D), jnp.float32),
                   jax.ShapeDtypeStruct((B, S, D), jnp.float32)],
    )(acc, set_feat, Wvc, dbvc, g_set, b_set, Wvs, bvs)


_TAIL_R = 1296


def _tc_tail_body(cf_ref, gsum_ref, gc_ref, bc_ref, w1_ref, b1_ref, w2_ref,
                  b2_ref, gf_ref, bf_ref, out_ref):
    x0 = cf_ref[...] + gsum_ref[...]
    c1 = _ln(x0, gc_ref[...], bc_ref[...])
    h = jnp.maximum(jnp.dot(c1, w1_ref[...], preferred_element_type=jnp.float32)
                    + b1_ref[...], 0.0)
    y = jnp.dot(h, w2_ref[...], preferred_element_type=jnp.float32) + b2_ref[...]
    out_ref[...] = _ln(c1 + y, gf_ref[...], bf_ref[...])


def _tc_tail(cf2, gsum2, g_cand, b_cand, W1, b1, W2, b2, g_ff, b_ff):
    rows = B * C
    grid = (rows // _TAIL_R,)
    bs = pl.BlockSpec((_TAIL_R, D), lambda i: (i, 0))
    w1s = pl.BlockSpec((D, 2 * D), lambda i: (0, 0))
    w2s = pl.BlockSpec((2 * D, D), lambda i: (0, 0))
    vec = pl.BlockSpec((D,), lambda i: (0,))
    vec2 = pl.BlockSpec((2 * D,), lambda i: (0,))
    return pl.pallas_call(
        _tc_tail_body,
        grid=grid,
        in_specs=[bs, bs, vec, vec, w1s, vec2, w2s, vec, vec, vec],
        out_specs=bs,
        out_shape=jax.ShapeDtypeStruct((rows, D), jnp.float32),
    )(cf2, gsum2, g_cand, b_cand, W1, b1, W2, b2, g_ff, b_ff)


def kernel(cand_feat, set_feat, cand_to_set_idx, Wvc, bvc, Wvs, bvs, g_set,
           b_set, g_cand, b_cand, W1, b1, W2, b2, g_ff, b_ff):
    idx = cand_to_set_idx.astype(jnp.int32)

    sidx = idx.T.reshape(3, NSCH, SCH)
    sidx_off = sidx[None] + (jnp.arange(NS, dtype=jnp.int32) * S)[:, None, None, None]
    gidx = idx.reshape(1, NGCH, 3 * GCH)
    gidx_abs = gidx + (jnp.arange(B, dtype=jnp.int32) * S)[:, None, None]
    deg = jnp.sum((idx.reshape(-1, 1) == jnp.arange(S, dtype=jnp.int32)[None, :])
                  .astype(jnp.float32), axis=0)
    dbvc = deg[:, None] * (bvc[None, :] * (1.0 / 9.0))
    zeros_sd = jnp.zeros((S, D), jnp.float32)

    acc = _sc_scatter(cand_feat, sidx_off, zeros_sd)
    set_out, proj3 = _tc_mid(acc, set_feat, Wvc, dbvc, g_set, b_set, Wvs, bvs)
    gsum = _sc_gather(proj3.reshape(B * S, D), gidx_abs)
    cand = _tc_tail(cand_feat.reshape(B * C, D), gsum.reshape(B * C, D),
                    g_cand, b_cand, W1, b1, W2, b2, g_ff, b_ff)
    return (cand.reshape(B, C, D), set_out)


# speedup vs baseline: 1.4015x; 1.1814x over previous
import functools

import jax
import jax.numpy as jnp
from jax import lax
from jax.experimental import pallas as pl
from jax.experimental.pallas import tpu as pltpu
from jax.experimental.pallas import tpu_sc as plsc

B, C, S, D = 256, 729, 243, 128
NC, NS = 2, 16
NW = NC * NS
BPW = B // NW
SCH = 81
NSCH = C // SCH
GCH = 27
NGCH = C // GCH
L = 16

_mesh = plsc.VectorSubcoreMesh(core_axis_name="c", subcore_axis_name="s",
                               num_cores=NC, num_subcores=NS)
_sc_params = pltpu.CompilerParams(use_tc_tiling_on_sc=False)


def _sc_scatter_body(cand_hbm, sidx_hbm, zeros_hbm, out_hbm, idx_v, src_v,
                     acc_sh, lsem, asem, osem):
    cid = lax.axis_index("c")
    sid = lax.axis_index("s")
    wid = sid * NC + cid
    pltpu.sync_copy(sidx_hbm.at[sid], idx_v)
    acc = acc_sh.at[pl.ds(sid * S, S)]

    def batch_body(bi, carry):
        b = wid * BPW + bi
        @pl.when(bi > 0)
        def _():
            pltpu.make_async_copy(acc, out_hbm.at[b - 1], osem).wait()
        pltpu.sync_copy(zeros_hbm, acc)
        loads = [None, None]
        prev_adds = [None, None]
        loads[0] = pltpu.async_copy(cand_hbm.at[b, pl.ds(0, SCH)],
                                    src_v.at[0], lsem.at[0])
        for ch in range(NSCH):
            s = ch % 2
            o = 1 - s
            loads[s].wait()
            if ch + 1 < NSCH:
                if prev_adds[o] is not None:
                    for h in prev_adds[o]:
                        h.wait()
                loads[o] = pltpu.async_copy(
                    cand_hbm.at[b, pl.ds((ch + 1) * SCH, SCH)],
                    src_v.at[o], lsem.at[o])
            prev_adds[s] = [
                pltpu.async_copy(src_v.at[s], acc_sh.at[idx_v.at[j, ch]],
                                 asem.at[s], add=True)
                for j in range(3)]
        for s in (0, 1):
            for h in prev_adds[s]:
                h.wait()
        pltpu.async_copy(acc, out_hbm.at[b], osem)
        return carry

    lax.fori_loop(0, BPW, batch_body, 0)
    pltpu.make_async_copy(acc, out_hbm.at[wid * BPW + BPW - 1], osem).wait()


_sc_scatter = pl.kernel(
    _sc_scatter_body,
    out_type=jax.ShapeDtypeStruct((B, S, D), jnp.float32),
    mesh=_mesh,
    compiler_params=_sc_params,
    scratch_types=[
        pltpu.VMEM((3, NSCH, SCH), jnp.int32),
        pltpu.VMEM((2, SCH, D), jnp.float32),
        pltpu.VMEM_SHARED((NS * S, D), jnp.float32),
        pltpu.SemaphoreType.DMA((2,)),
        pltpu.SemaphoreType.DMA((2,)),
        pltpu.SemaphoreType.DMA,
    ],
)


_GT = BPW * NGCH


def _sc_gather_body(proj_hbm, gidx_hbm, out_hbm, idx_v, gbuf0, gbuf1, obuf0,
                    obuf1, gsem, osem):
    cid = lax.axis_index("c")
    sid = lax.axis_index("s")
    wid = sid * NC + cid
    pltpu.sync_copy(gidx_hbm.at[pl.ds(wid * _GT, _GT)], idx_v)
    gbufs = (gbuf0, gbuf1)
    obufs = (obuf0, obuf1)

    def _issue(t, s):
        return pltpu.async_copy(proj_hbm.at[idx_v.at[t]], gbufs[s], gsem.at[s])

    def _out_dst(t):
        bi = t // NGCH
        ch = t - bi * NGCH
        return out_hbm.at[wid * BPW + bi, pl.ds(ch * GCH, GCH)]

    _issue(0, 0)

    def pair_body(tt, carry):
        t0 = 2 * tt
        for s in (0, 1):
            t = t0 + s
            pltpu.make_async_copy(proj_hbm.at[idx_v.at[t]], gbufs[s],
                                  gsem.at[s]).wait()
            @pl.when(t + 1 < _GT)
            def _():
                _issue(t + 1, 1 - s)
            @pl.when(t >= 2)
            def _():
                pltpu.make_async_copy(obufs[s], _out_dst(t - 2),
                                      osem.at[s]).wait()
            for i in range(GCH):
                for k in range(D // L):
                    sl = pl.ds(k * L, L)
                    obufs[s][i, sl] = (gbufs[s][3 * i, sl]
                                       + gbufs[s][3 * i + 1, sl]
                                       + gbufs[s][3 * i + 2, sl])
            pltpu.async_copy(obufs[s], _out_dst(t), osem.at[s])
        return carry

    lax.fori_loop(0, _GT // 2, pair_body, 0)
    for s in (0, 1):
        pltpu.make_async_copy(obufs[s], _out_dst(_GT - 2 + s),
                              osem.at[s]).wait()


_sc_gather = pl.kernel(
    _sc_gather_body,
    out_type=jax.ShapeDtypeStruct((B, C, D), jnp.float32),
    mesh=_mesh,
    compiler_params=_sc_params,
    scratch_types=[
        pltpu.VMEM((_GT, 3 * GCH), jnp.int32),
        pltpu.VMEM((3 * GCH, D), jnp.float32),
        pltpu.VMEM((3 * GCH, D), jnp.float32),
        pltpu.VMEM((GCH, D), jnp.float32),
        pltpu.VMEM((GCH, D), jnp.float32),
        pltpu.SemaphoreType.DMA((2,)),
        pltpu.SemaphoreType.DMA((2,)),
    ],
)


def _ln(x, g, b, eps=1e-5):
    m = x.mean(-1, keepdims=True)
    v = ((x - m) ** 2).mean(-1, keepdims=True)
    return (x - m) / jnp.sqrt(v + eps) * g + b


_MID_BB = 4


def _tc_mid_body(acc_ref, sf_ref, wvc_ref, dbvc_ref, gs_ref, bs_ref, wvs_ref,
                 bvs_ref, so_ref, pr_ref):
    wvc = wvc_ref[...]
    wvs = wvs_ref[...]
    dbvc = dbvc_ref[...]
    for i in range(_MID_BB):
        su = jnp.dot(acc_ref[i], wvc, preferred_element_type=jnp.float32) * (1.0 / 9.0)
        x = sf_ref[i] + su + dbvc
        so = _ln(x, gs_ref[...], bs_ref[...])
        so_ref[i] = so
        pr_ref[i] = (jnp.dot(so, wvs, preferred_element_type=jnp.float32) * (1.0 / 3.0)
                     + bvs_ref[...] * (1.0 / 3.0))


def _tc_mid(acc, set_feat, Wvc, dbvc, g_set, b_set, Wvs, bvs):
    grid = (B // _MID_BB,)
    bs3 = pl.BlockSpec((_MID_BB, S, D), lambda i: (i, 0, 0))
    full2 = pl.BlockSpec((S, D), lambda i: (0, 0))
    sq = pl.BlockSpec((D, D), lambda i: (0, 0))
    vec = pl.BlockSpec((D,), lambda i: (0,))
    return pl.pallas_call(
        _tc_mid_body,
        grid=grid,
        in_specs=[bs3, bs3, sq, full2, vec, vec, sq, vec],
        out_specs=[bs3, bs3],
        out_shape=[jax.ShapeDtypeStruct((B, S, D), jnp.float32),
                   jax.ShapeDtypeStruct((B, S, D), jnp.float32)],
    )(acc, set_feat, Wvc, dbvc, g_set, b_set, Wvs, bvs)


_TAIL_R = 1296


def _tc_tail_body(cf_ref, gsum_ref, gc_ref, bc_ref, w1_ref, b1_ref, w2_ref,
                  b2_ref, gf_ref, bf_ref, out_ref):
    x0 = cf_ref[...] + gsum_ref[...]
    c1 = _ln(x0, gc_ref[...], bc_ref[...])
    h = jnp.maximum(jnp.dot(c1, w1_ref[...], preferred_element_type=jnp.float32)
                    + b1_ref[...], 0.0)
    y = jnp.dot(h, w2_ref[...], preferred_element_type=jnp.float32) + b2_ref[...]
    out_ref[...] = _ln(c1 + y, gf_ref[...], bf_ref[...])


def _tc_tail(cf2, gsum2, g_cand, b_cand, W1, b1, W2, b2, g_ff, b_ff):
    rows = B * C
    grid = (rows // _TAIL_R,)
    bs = pl.BlockSpec((_TAIL_R, D), lambda i: (i, 0))
    w1s = pl.BlockSpec((D, 2 * D), lambda i: (0, 0))
    w2s = pl.BlockSpec((2 * D, D), lambda i: (0, 0))
    vec = pl.BlockSpec((D,), lambda i: (0,))
    vec2 = pl.BlockSpec((2 * D,), lambda i: (0,))
    return pl.pallas_call(
        _tc_tail_body,
        grid=grid,
        in_specs=[bs, bs, vec, vec, w1s, vec2, w2s, vec, vec, vec],
        out_specs=bs,
        out_shape=jax.ShapeDtypeStruct((rows, D), jnp.float32),
    )(cf2, gsum2, g_cand, b_cand, W1, b1, W2, b2, g_ff, b_ff)


def kernel(cand_feat, set_feat, cand_to_set_idx, Wvc, bvc, Wvs, bvs, g_set,
           b_set, g_cand, b_cand, W1, b1, W2, b2, g_ff, b_ff):
    idx = cand_to_set_idx.astype(jnp.int32)

    sidx = idx.T.reshape(3, NSCH, SCH)
    sidx_off = sidx[None] + (jnp.arange(NS, dtype=jnp.int32) * S)[:, None, None, None]
    gidx = idx.reshape(1, NGCH, 3 * GCH)
    gidx_abs = gidx + (jnp.arange(B, dtype=jnp.int32) * S)[:, None, None]
    deg = jnp.sum((idx.reshape(-1, 1) == jnp.arange(S, dtype=jnp.int32)[None, :])
                  .astype(jnp.float32), axis=0)
    dbvc = deg[:, None] * (bvc[None, :] * (1.0 / 9.0))
    zeros_sd = jnp.zeros((S, D), jnp.float32)

    acc = _sc_scatter(cand_feat, sidx_off, zeros_sd)
    set_out, proj3 = _tc_mid(acc, set_feat, Wvc, dbvc, g_set, b_set, Wvs, bvs)
    gsum = _sc_gather(proj3.reshape(B * S, D),
                      gidx_abs.reshape(B * NGCH, 3 * GCH))
    cand = _tc_tail(cand_feat.reshape(B * C, D), gsum.reshape(B * C, D),
                    g_cand, b_cand, W1, b1, W2, b2, g_ff, b_ff)
    return (cand.reshape(B, C, D), set_out)


# speedup vs baseline: 1.4698x; 1.0488x over previous
import functools

import jax
import jax.numpy as jnp
from jax import lax
from jax.experimental import pallas as pl
from jax.experimental.pallas import tpu as pltpu
from jax.experimental.pallas import tpu_sc as plsc

B, C, S, D = 256, 729, 243, 128
NC, NS = 2, 16
NW = NC * NS
BPW = B // NW
SCH = 81
NSCH = C // SCH
GCH = 27
NGCH = C // GCH
L = 16

_mesh = plsc.VectorSubcoreMesh(core_axis_name="c", subcore_axis_name="s",
                               num_cores=NC, num_subcores=NS)


def _sc_scatter_body(cand_hbm, sidx_hbm, lidx_hbm, zeros_hbm, out_hbm, idx_v,
                     lidx_v, src_v, acc_sh, lsem, asem, osem):
    cid = lax.axis_index("c")
    sid = lax.axis_index("s")
    wid = sid * NC + cid
    pltpu.sync_copy(sidx_hbm.at[sid], idx_v)
    pltpu.sync_copy(lidx_hbm.at[pl.ds(wid * BPW * NSCH, BPW * NSCH)], lidx_v)
    acc = acc_sh.at[pl.ds(sid * S, S)]

    def _load(bi, ch, s):
        return pltpu.async_copy(cand_hbm.at[lidx_v.at[bi * NSCH + ch]],
                                src_v.at[s], lsem.at[s])

    def batch_body(bi, carry):
        b = wid * BPW + bi
        @pl.when(bi > 0)
        def _():
            pltpu.make_async_copy(acc, out_hbm.at[b - 1], osem).wait()
        pltpu.sync_copy(zeros_hbm, acc)
        loads = [None, None]
        prev_adds = [None, None]
        loads[0] = _load(bi, 0, 0)
        for ch in range(NSCH):
            s = ch % 2
            o = 1 - s
            loads[s].wait()
            if ch + 1 < NSCH:
                if prev_adds[o] is not None:
                    for h in prev_adds[o]:
                        h.wait()
                loads[o] = _load(bi, ch + 1, o)
            prev_adds[s] = [
                pltpu.async_copy(src_v.at[s], acc_sh.at[idx_v.at[j, ch]],
                                 asem.at[s], add=True)
                for j in range(3)]
        for s in (0, 1):
            for h in prev_adds[s]:
                h.wait()
        pltpu.async_copy(acc, out_hbm.at[b], osem)
        return carry

    lax.fori_loop(0, BPW, batch_body, 0)
    pltpu.make_async_copy(acc, out_hbm.at[wid * BPW + BPW - 1], osem).wait()


_sc_scatter = pl.kernel(
    _sc_scatter_body,
    out_type=jax.ShapeDtypeStruct((B, S, D), jnp.float32),
    mesh=_mesh,
    scratch_types=[
        pltpu.VMEM((3, NSCH, SCH), jnp.int32),
        pltpu.VMEM((BPW * NSCH, SCH), jnp.int32),
        pltpu.VMEM((2, SCH, D), jnp.float32),
        pltpu.VMEM_SHARED((NS * S, D), jnp.float32),
        pltpu.SemaphoreType.DMA((2,)),
        pltpu.SemaphoreType.DMA((2,)),
        pltpu.SemaphoreType.DMA,
    ],
)


_GT = BPW * NGCH


def _sc_gather_body(proj_hbm, gidx_hbm, oidx_hbm, out_hbm, idx_v, oidx_v,
                    gbuf0, gbuf1, obuf0, obuf1, gsem, osem):
    cid = lax.axis_index("c")
    sid = lax.axis_index("s")
    wid = sid * NC + cid
    pltpu.sync_copy(gidx_hbm.at[pl.ds(wid * _GT, _GT)], idx_v)
    pltpu.sync_copy(oidx_hbm.at[pl.ds(wid * _GT, _GT)], oidx_v)
    gbufs = (gbuf0, gbuf1)
    obufs = (obuf0, obuf1)

    def _issue(t, s):
        return pltpu.async_copy(proj_hbm.at[idx_v.at[t]], gbufs[s], gsem.at[s])

    def _out_dst(t):
        return out_hbm.at[oidx_v.at[t]]

    _issue(0, 0)

    def pair_body(tt, carry):
        t0 = 2 * tt
        for s in (0, 1):
            t = t0 + s
            pltpu.make_async_copy(proj_hbm.at[idx_v.at[t]], gbufs[s],
                                  gsem.at[s]).wait()
            @pl.when(t + 1 < _GT)
            def _():
                _issue(t + 1, 1 - s)
            @pl.when(t >= 2)
            def _():
                pltpu.make_async_copy(obufs[s], _out_dst(t - 2),
                                      osem.at[s]).wait()
            for i in range(GCH):
                for k in range(D // L):
                    sl = pl.ds(k * L, L)
                    obufs[s][i, sl] = (gbufs[s][3 * i, sl]
                                       + gbufs[s][3 * i + 1, sl]
                                       + gbufs[s][3 * i + 2, sl])
            pltpu.async_copy(obufs[s], _out_dst(t), osem.at[s])
        return carry

    lax.fori_loop(0, _GT // 2, pair_body, 0)
    for s in (0, 1):
        pltpu.make_async_copy(obufs[s], _out_dst(_GT - 2 + s),
                              osem.at[s]).wait()


_sc_gather = pl.kernel(
    _sc_gather_body,
    out_type=jax.ShapeDtypeStruct((B * C, D), jnp.float32),
    mesh=_mesh,
    scratch_types=[
        pltpu.VMEM((_GT, 3 * GCH), jnp.int32),
        pltpu.VMEM((_GT, GCH), jnp.int32),
        pltpu.VMEM((3 * GCH, D), jnp.float32),
        pltpu.VMEM((3 * GCH, D), jnp.float32),
        pltpu.VMEM((GCH, D), jnp.float32),
        pltpu.VMEM((GCH, D), jnp.float32),
        pltpu.SemaphoreType.DMA((2,)),
        pltpu.SemaphoreType.DMA((2,)),
    ],
)


def _ln(x, g, b, eps=1e-5):
    m = x.mean(-1, keepdims=True)
    v = ((x - m) ** 2).mean(-1, keepdims=True)
    return (x - m) / jnp.sqrt(v + eps) * g + b


_MID_BB = 4


def _tc_mid_body(acc_ref, sf_ref, wvc_ref, dbvc_ref, gs_ref, bs_ref, wvs_ref,
                 bvs_ref, so_ref, pr_ref):
    wvc = wvc_ref[...]
    wvs = wvs_ref[...]
    dbvc = dbvc_ref[...]
    for i in range(_MID_BB):
        su = jnp.dot(acc_ref[i], wvc, preferred_element_type=jnp.float32) * (1.0 / 9.0)
        x = sf_ref[i] + su + dbvc
        so = _ln(x, gs_ref[...], bs_ref[...])
        so_ref[i] = so
        pr_ref[i] = (jnp.dot(so, wvs, preferred_element_type=jnp.float32) * (1.0 / 3.0)
                     + bvs_ref[...] * (1.0 / 3.0))


def _tc_mid(acc, set_feat, Wvc, dbvc, g_set, b_set, Wvs, bvs):
    grid = (B // _MID_BB,)
    bs3 = pl.BlockSpec((_MID_BB, S, D), lambda i: (i, 0, 0))
    full2 = pl.BlockSpec((S, D), lambda i: (0, 0))
    sq = pl.BlockSpec((D, D), lambda i: (0, 0))
    vec = pl.BlockSpec((D,), lambda i: (0,))
    return pl.pallas_call(
        _tc_mid_body,
        grid=grid,
        in_specs=[bs3, bs3, sq, full2, vec, vec, sq, vec],
        out_specs=[bs3, bs3],
        out_shape=[jax.ShapeDtypeStruct((B, S, D), jnp.float32),
                   jax.ShapeDtypeStruct((B, S, D), jnp.float32)],
    )(acc, set_feat, Wvc, dbvc, g_set, b_set, Wvs, bvs)


_TAIL_R = 1296


def _tc_tail_body(cf_ref, gsum_ref, gc_ref, bc_ref, w1_ref, b1_ref, w2_ref,
                  b2_ref, gf_ref, bf_ref, out_ref):
    x0 = cf_ref[...] + gsum_ref[...]
    c1 = _ln(x0, gc_ref[...], bc_ref[...])
    h = jnp.maximum(jnp.dot(c1, w1_ref[...], preferred_element_type=jnp.float32)
                    + b1_ref[...], 0.0)
    y = jnp.dot(h, w2_ref[...], preferred_element_type=jnp.float32) + b2_ref[...]
    out_ref[...] = _ln(c1 + y, gf_ref[...], bf_ref[...])


def _tc_tail(cf2, gsum2, g_cand, b_cand, W1, b1, W2, b2, g_ff, b_ff):
    rows = B * C
    grid = (rows // _TAIL_R,)
    bs = pl.BlockSpec((_TAIL_R, D), lambda i: (i, 0))
    w1s = pl.BlockSpec((D, 2 * D), lambda i: (0, 0))
    w2s = pl.BlockSpec((2 * D, D), lambda i: (0, 0))
    vec = pl.BlockSpec((D,), lambda i: (0,))
    vec2 = pl.BlockSpec((2 * D,), lambda i: (0,))
    return pl.pallas_call(
        _tc_tail_body,
        grid=grid,
        in_specs=[bs, bs, vec, vec, w1s, vec2, w2s, vec, vec, vec],
        out_specs=bs,
        out_shape=jax.ShapeDtypeStruct((rows, D), jnp.float32),
    )(cf2, gsum2, g_cand, b_cand, W1, b1, W2, b2, g_ff, b_ff)


def kernel(cand_feat, set_feat, cand_to_set_idx, Wvc, bvc, Wvs, bvs, g_set,
           b_set, g_cand, b_cand, W1, b1, W2, b2, g_ff, b_ff):
    idx = cand_to_set_idx.astype(jnp.int32)

    sidx = idx.T.reshape(3, NSCH, SCH)
    sidx_off = sidx[None] + (jnp.arange(NS, dtype=jnp.int32) * S)[:, None, None, None]
    gidx = idx.reshape(1, NGCH, 3 * GCH)
    gidx_abs = gidx + (jnp.arange(B, dtype=jnp.int32) * S)[:, None, None]
    deg = jnp.sum((idx.reshape(-1, 1) == jnp.arange(S, dtype=jnp.int32)[None, :])
                  .astype(jnp.float32), axis=0)
    dbvc = deg[:, None] * (bvc[None, :] * (1.0 / 9.0))
    zeros_sd = jnp.zeros((S, D), jnp.float32)
    brow = jnp.arange(B, dtype=jnp.int32)[:, None, None] * C
    lidx = (brow + jnp.arange(NSCH, dtype=jnp.int32)[None, :, None] * SCH
            + jnp.arange(SCH, dtype=jnp.int32)[None, None, :]
            ).reshape(B * NSCH, SCH)
    oidx = (brow + jnp.arange(NGCH, dtype=jnp.int32)[None, :, None] * GCH
            + jnp.arange(GCH, dtype=jnp.int32)[None, None, :]
            ).reshape(B * NGCH, GCH)

    acc = _sc_scatter(cand_feat.reshape(B * C, D), sidx_off, lidx, zeros_sd)
    set_out, proj3 = _tc_mid(acc, set_feat, Wvc, dbvc, g_set, b_set, Wvs, bvs)
    gsum = _sc_gather(proj3.reshape(B * S, D),
                      gidx_abs.reshape(B * NGCH, 3 * GCH), oidx)
    cand = _tc_tail(cand_feat.reshape(B * C, D), gsum,
                    g_cand, b_cand, W1, b1, W2, b2, g_ff, b_ff)
    return (cand.reshape(B, C, D), set_out)
